# Initial kernel scaffold; baseline (speedup 1.0000x reference)
import jax
import jax.numpy as jnp
from jax.experimental import pallas as pl


def kernel(feats, edge_index, node_graph_ids, W_in, b_in, W_ih, W_hh, b_ih, b_hh, bn_gamma, bn_beta, Wg, bg, W_if, W_hf, b_if, b_hf, W_ib, W_hb, b_ib, b_hb, Wo, bo):
    raise NotImplementedError("write your pallas kernel here")


# speedup vs baseline: 4.0188x; 4.0188x over previous
import functools

import jax
import jax.numpy as jnp
from jax import lax
from jax.experimental import pallas as pl
from jax.experimental.pallas import tpu as pltpu
from jax.experimental.pallas import tpu_sc as plsc

N = 10000
E = 320000
D = 128
B = 256
NC_OUT = 12
EPS_GIN = 1e-05
BN_EPS = 1e-05

NSC = 2
NTEC = 16
NW = NSC * NTEC

NPT = 320
NP = NW * NPT
RC = 64
NRC = NPT // RC

CH = 80
EPT = E // NW
NCHE = EPT // CH

BP = 264

AGG_ROWS = N + 8
AGG_TILE = 640
AGG_CH = 80
SG_CH = 24

BLK = 1024

_SC_MESH = plsc.VectorSubcoreMesh(
    core_axis_name="c", subcore_axis_name="s", num_cores=NSC,
    num_subcores=NTEC)


def _sc_body(with_edges, *refs):
  if with_edges:
    (f, u, h, src, dst, ids, zhbm, ninf,
     aggp, sumgp, maxgp,
     agg_acc, sumg_acc, stage, sidx, didx,
     ridx, idsbuf, maxbuf, sem) = refs
  else:
    (u, h, ids, zhbm, ninf,
     sumgp, maxgp,
     sumg_acc, stage,
     ridx, idsbuf, maxbuf, sem) = refs

  cid = lax.axis_index("c")
  tid = lax.axis_index("s")
  wid = cid * NTEC + tid

  pltpu.sync_copy(zhbm, stage)
  pltpu.sync_copy(ninf, maxbuf)
  nch = jnp.where(tid < NTEC - 1, 8, 5)
  if with_edges:
    def zbody(k, carry):
      pltpu.sync_copy(stage.at[pl.ds(0, AGG_CH)],
                      agg_acc.at[pl.ds(tid * AGG_TILE + k * AGG_CH,
                                       AGG_CH)])
      return carry

    lax.fori_loop(0, nch, zbody, 0)

    @pl.when(tid < 3)
    def _():
      pltpu.sync_copy(stage.at[pl.ds(0, 80)],
                      aggp.at[cid, pl.ds(N + tid * 80, 80)])

  @pl.when(tid < 11)
  def _():
    pltpu.sync_copy(stage.at[pl.ds(0, SG_CH)],
                    sumg_acc.at[pl.ds(tid * SG_CH, SG_CH)])

  plsc.subcore_barrier()

  if with_edges:
    ebase = wid * EPT

    def echunk(c, carry):
      b = ebase + c * CH
      pltpu.sync_copy(src.at[pl.ds(b, CH)], sidx)
      pltpu.sync_copy(dst.at[pl.ds(b, CH)], didx)
      pltpu.async_copy(f.at[sidx], stage, sem).wait()
      pltpu.sync_copy(stage, agg_acc.at[didx], add=True)
      return carry

    lax.fori_loop(0, NCHE, echunk, 0)

  nbase = wid * NPT
  pltpu.sync_copy(ids.at[pl.ds(nbase, NPT)], idsbuf.at[pl.ds(0, NPT)])
  for k in range(NRC):
    off = nbase + k * RC
    pltpu.sync_copy(u.at[pl.ds(off, RC)], stage.at[pl.ds(0, RC)])
    pltpu.sync_copy(ids.at[pl.ds(off, RC)], ridx)
    pltpu.sync_copy(stage.at[pl.ds(0, RC)], sumg_acc.at[ridx], add=True)
    pltpu.sync_copy(h.at[pl.ds(off, RC)], stage.at[pl.ds(0, RC)])

    def mbody(n, carry, k=k):
      g = idsbuf[pl.ds(k * RC + n, 16)][0]
      for j in range(D // 16):
        sl = pl.ds(j * 16, 16)
        maxbuf[g, sl] = jnp.maximum(maxbuf[g, sl], stage[n, sl])
      return carry

    lax.fori_loop(0, RC, mbody, 0)

  plsc.subcore_barrier()

  pltpu.sync_copy(maxbuf, maxgp.at[wid])
  if with_edges:
    def wbody(k, carry):
      r0 = tid * AGG_TILE + k * AGG_CH
      pltpu.sync_copy(agg_acc.at[pl.ds(r0, AGG_CH)],
                      aggp.at[cid, pl.ds(r0, AGG_CH)])
      return carry

    lax.fori_loop(0, nch, wbody, 0)

  @pl.when(tid < 11)
  def _():
    pltpu.sync_copy(sumg_acc.at[pl.ds(tid * SG_CH, SG_CH)],
                    sumgp.at[cid, pl.ds(tid * SG_CH, SG_CH)])


def _make_sc_call(with_edges):
  f32 = jnp.float32
  if with_edges:
    out_type = [
        jax.ShapeDtypeStruct((NSC, NP, D), f32),
        jax.ShapeDtypeStruct((NSC, BP, D), f32),
        jax.ShapeDtypeStruct((NW, BP, D), f32),
    ]
    scratch = [
        pltpu.VMEM_SHARED((AGG_ROWS, D), f32),
        pltpu.VMEM_SHARED((BP, D), f32),
        pltpu.VMEM((CH, D), f32),
        pltpu.VMEM((CH,), jnp.int32),
        pltpu.VMEM((CH,), jnp.int32),
        pltpu.VMEM((RC,), jnp.int32),
        pltpu.VMEM((NPT + 16,), jnp.int32),
        pltpu.VMEM((BP, D), f32),
        pltpu.SemaphoreType.DMA,
    ]
  else:
    out_type = [
        jax.ShapeDtypeStruct((NSC, BP, D), f32),
        jax.ShapeDtypeStruct((NW, BP, D), f32),
    ]
    scratch = [
        pltpu.VMEM_SHARED((BP, D), f32),
        pltpu.VMEM((CH, D), f32),
        pltpu.VMEM((RC,), jnp.int32),
        pltpu.VMEM((NPT + 16,), jnp.int32),
        pltpu.VMEM((BP, D), f32),
        pltpu.SemaphoreType.DMA,
    ]
  return pl.kernel(
      functools.partial(_sc_body, with_edges),
      out_type=out_type,
      mesh=_SC_MESH,
      scratch_types=scratch,
      name="gin_sc_edges" if with_edges else "gin_sc_readout",
  )


_sc_edges = _make_sc_call(True)
_sc_readout = _make_sc_call(False)


def _row_spec(blk):
  return pl.BlockSpec((blk, D), lambda i: (i, 0))


def _full(shape):
  nd = len(shape)
  return pl.BlockSpec(shape, lambda i, nd=nd: (0,) * nd)


def _init_body(x_ref, wt_ref, bin_ref, g_ref, bt_ref, wg_ref, bg_ref,
               h_ref, f_ref, u_ref):
  x = x_ref[...]
  h = jnp.dot(x, wt_ref[...], preferred_element_type=jnp.float32)
  h = h + bin_ref[...]
  h_ref[...] = h
  f_ref[...] = h * g_ref[...] + bt_ref[...]
  wl = jnp.sum(h * wg_ref[...], axis=1, keepdims=True) + bg_ref[...]
  u_ref[...] = jax.nn.sigmoid(wl) * h


def _gru_body(f_ref, a_ref, h_ref, wa_ref, wb_ref, wh_ref, bih_ref,
              bhh_ref, g_ref, bt_ref, wg_ref, bg_ref,
              hn_ref, fn_ref, un_ref):
  f = f_ref[...]
  agg = a_ref[0] + a_ref[1]
  h = h_ref[...]
  fs = f * (1.0 + EPS_GIN)
  gi = jnp.dot(fs, wa_ref[...], preferred_element_type=jnp.float32)
  gi = gi + jnp.dot(agg, wb_ref[...], preferred_element_type=jnp.float32)
  gi = gi + bih_ref[...]
  gh = jnp.dot(f, wh_ref[...], preferred_element_type=jnp.float32)
  gh = gh + bhh_ref[...]
  ir, iz, inn = gi[:, :D], gi[:, D:2 * D], gi[:, 2 * D:]
  hr, hz, hn = gh[:, :D], gh[:, D:2 * D], gh[:, 2 * D:]
  r = jax.nn.sigmoid(ir + hr)
  z = jax.nn.sigmoid(iz + hz)
  n = jnp.tanh(inn + r * hn)
  new = (1.0 - z) * n + z * f
  new = jnp.where(new > 0, new, jnp.exp(jnp.minimum(new, 0.0)) - 1.0)
  hn2 = new + h
  hn_ref[...] = hn2
  fn_ref[...] = hn2 * g_ref[...] + bt_ref[...]
  wl = jnp.sum(hn2 * wg_ref[...], axis=1, keepdims=True) + bg_ref[...]
  un_ref[...] = jax.nn.sigmoid(wl) * hn2


def _r1_body(sp_ref, mp_ref, out_ref):
  s = sp_ref[0, 0] + sp_ref[0, 1]
  m = jnp.max(mp_ref[0], axis=0)
  m = jnp.where(jnp.isneginf(m), 0.0, m)
  out_ref[0] = jnp.concatenate([s[:B], m[:B]], axis=-1)


def _r2_body(seq_ref, wif_ref, bif_ref, wib_ref, whb_ref, bib_ref,
             bhb_ref, bhf_ref, wo_ref, bo_ref, out_ref):
  x0 = seq_ref[0]
  g = jnp.dot(x0, wif_ref[...], preferred_element_type=jnp.float32)
  g = g + bif_ref[...] + bhf_ref[...]
  i, fgate, gg, o = (g[:, :D], g[:, D:2 * D], g[:, 2 * D:3 * D],
                     g[:, 3 * D:])
  c = jax.nn.sigmoid(i) * jnp.tanh(gg)
  hf0 = jax.nn.sigmoid(o) * jnp.tanh(c)
  hb = jnp.zeros((B, D), jnp.float32)
  cb = jnp.zeros((B, D), jnp.float32)
  for t in (2, 1, 0):
    x = seq_ref[t]
    g = jnp.dot(x, wib_ref[...], preferred_element_type=jnp.float32)
    g = g + bib_ref[...]
    g = g + jnp.dot(hb, whb_ref[...], preferred_element_type=jnp.float32)
    g = g + bhb_ref[...]
    i, fgate, gg, o = (g[:, :D], g[:, D:2 * D], g[:, 2 * D:3 * D],
                       g[:, 3 * D:])
    cb = jax.nn.sigmoid(fgate) * cb + jax.nn.sigmoid(i) * jnp.tanh(gg)
    hb = jax.nn.sigmoid(o) * jnp.tanh(cb)
  out = jnp.concatenate([hf0, hb], axis=-1)
  out = jnp.dot(out, wo_ref[...], preferred_element_type=jnp.float32)
  out_ref[...] = out + bo_ref[...]


def _node_grid_call(body, n_extra_in, n_out, extra_shapes):
  grid = (NP // BLK,)
  in_specs = extra_shapes
  return pl.pallas_call(
      body,
      grid=grid,
      in_specs=in_specs,
      out_specs=[_row_spec(BLK)] * n_out,
      out_shape=[jax.ShapeDtypeStruct((NP, D), jnp.float32)] * n_out,
  )


def kernel(feats, edge_index, node_graph_ids, W_in, b_in, W_ih, W_hh,
           b_ih, b_hh, bn_gamma, bn_beta, Wg, bg, W_if, W_hf, b_if,
           b_hf, W_ib, W_hb, b_ib, b_hb, Wo, bo):
  f32 = jnp.float32
  inv = 1.0 / jnp.sqrt(jnp.asarray(1.0 + BN_EPS, f32))

  feats_p = jnp.pad(feats, ((0, NP - N), (0, 0)))
  ids_p = jnp.pad(node_graph_ids, (0, NP - N), constant_values=B)
  src = edge_index[0]
  dst = edge_index[1]
  zeros128 = jnp.zeros((CH, D), f32)
  neginf = jnp.full((BP, D), -jnp.inf, f32)

  wt_in = W_in.T
  b_in2 = b_in.reshape(1, D)
  wa = W_ih[:, :D].T
  wb = W_ih[:, D:].T
  wh = W_hh.T
  bih = b_ih.reshape(1, 3 * D)
  bhh = b_hh.reshape(1, 3 * D)
  gam = (bn_gamma * inv).astype(f32)
  bet = bn_beta
  wg_row = Wg.reshape(1, D)
  bg2 = bg.reshape(1, 1)

  wif = W_if.T
  bif = b_if.reshape(1, 4 * D)
  bhf = b_hf.reshape(1, 4 * D)
  wib = W_ib.T
  whb = W_hb.T
  bib = b_ib.reshape(1, 4 * D)
  bhb = b_hb.reshape(1, 4 * D)
  wo_p = jnp.zeros((2 * D, 128), f32).at[:, :NC_OUT].set(Wo.T)
  bo_p = jnp.zeros((1, 128), f32).at[0, :NC_OUT].set(bo)

  init_call = pl.pallas_call(
      _init_body,
      grid=(NP // BLK,),
      in_specs=[
          _row_spec(BLK), _full((D, D)), _full((1, D)), _full((1, D)),
          _full((1, D)), _full((1, D)), _full((1, 1)),
      ],
      out_specs=[_row_spec(BLK)] * 3,
      out_shape=[jax.ShapeDtypeStruct((NP, D), f32)] * 3,
  )
  h0, f0, u0 = init_call(feats_p, wt_in, b_in2, gam[0].reshape(1, D),
                         bet[0].reshape(1, D), wg_row, bg2)

  gru_call = pl.pallas_call(
      _gru_body,
      grid=(NP // BLK,),
      in_specs=[
          _row_spec(BLK),
          pl.BlockSpec((NSC, BLK, D), lambda i: (0, i, 0)),
          _row_spec(BLK),
          _full((D, 3 * D)), _full((D, 3 * D)), _full((D, 3 * D)),
          _full((1, 3 * D)), _full((1, 3 * D)),
          _full((1, D)), _full((1, D)), _full((1, D)), _full((1, 1)),
      ],
      out_specs=[_row_spec(BLK)] * 3,
      out_shape=[jax.ShapeDtypeStruct((NP, D), f32)] * 3,
  )

  aggp0, sg0, mg0 = _sc_edges(f0, u0, h0, src, dst, ids_p, zeros128,
                              neginf)
  h1, f1, u1 = gru_call(f0, aggp0, h0, wa, wb, wh, bih, bhh,
                        gam[1].reshape(1, D), bet[1].reshape(1, D),
                        wg_row, bg2)
  aggp1, sg1, mg1 = _sc_edges(f1, u1, h1, src, dst, ids_p, zeros128,
                              neginf)
  h2, _, u2 = gru_call(f1, aggp1, h1, wa, wb, wh, bih, bhh,
                       gam[2].reshape(1, D), bet[2].reshape(1, D),
                       wg_row, bg2)

  sg2, mg2 = _sc_readout(u2, h2, ids_p, zeros128, neginf)

  sum_all = jnp.stack([sg0, sg1, sg2])
  max_all = jnp.stack([mg0, mg1, mg2])

  r1_call = pl.pallas_call(
      _r1_body,
      grid=(3,),
      in_specs=[
          pl.BlockSpec((1, NSC, BP, D), lambda i: (i, 0, 0, 0)),
          pl.BlockSpec((1, NW, BP, D), lambda i: (i, 0, 0, 0)),
      ],
      out_specs=pl.BlockSpec((1, B, 2 * D), lambda i: (i, 0, 0)),
      out_shape=jax.ShapeDtypeStruct((3, B, 2 * D), f32),
  )
  seq = r1_call(sum_all, max_all)

  r2_call = pl.pallas_call(
      _r2_body,
      grid=(1,),
      in_specs=[
          _full((3, B, 2 * D)),
          _full((2 * D, 4 * D)), _full((1, 4 * D)),
          _full((2 * D, 4 * D)), _full((D, 4 * D)), _full((1, 4 * D)),
          _full((1, 4 * D)), _full((1, 4 * D)),
          _full((2 * D, 128)), _full((1, 128)),
      ],
      out_specs=_full((B, 128)),
      out_shape=jax.ShapeDtypeStruct((B, 128), f32),
  )
  out = r2_call(seq, wif, bif, wib, whb, bib, bhb, bhf, wo_p, bo_p)
  return out[:, :NC_OUT]


# speedup vs baseline: 7.5432x; 1.8770x over previous
import functools

import jax
import jax.numpy as jnp
from jax import lax
from jax.experimental import pallas as pl
from jax.experimental.pallas import tpu as pltpu
from jax.experimental.pallas import tpu_sc as plsc

N = 10000
E = 320000
D = 128
B = 256
NC_OUT = 12
EPS_GIN = 1e-05
BN_EPS = 1e-05

NSC = 2
NTEC = 16
NW = NSC * NTEC

NPT = 320
NP = NW * NPT
RC = 64
NRC = NPT // RC

CH = 80
EPT = E // NW
NCHE = EPT // CH
NBUF = 2

BP = 264
NL = 3
BP3 = NL * BP

AGG_ROWS = N + 8
AGG_TILE = 640
AGG_CH = 80
SG_CH = 72

BLK = 1024

_SC_MESH = plsc.VectorSubcoreMesh(
    core_axis_name="c", subcore_axis_name="s", num_cores=NSC,
    num_subcores=NTEC)


def _sc_edges_body(f, src1, dst2, zhbm, aggp,
                   agg_acc, sblk, dblk, st0, st1, sem0, sem1):
  cid = lax.axis_index("c")
  tid = lax.axis_index("s")
  wid = cid * NTEC + tid
  stages = (st0, st1)
  sems = (sem0, sem1)

  pltpu.sync_copy(src1.at[pl.ds(wid * EPT, EPT)], sblk)
  pltpu.sync_copy(dst2.at[wid], dblk)
  pltpu.sync_copy(zhbm, st0)
  nch = jnp.where(tid < NTEC - 1, 8, 5)

  def zbody(k, carry):
    pltpu.sync_copy(st0.at[pl.ds(0, AGG_CH)],
                    agg_acc.at[pl.ds(tid * AGG_TILE + k * AGG_CH, AGG_CH)])
    return carry

  lax.fori_loop(0, nch, zbody, 0)

  @pl.when(tid < 3)
  def _():
    pltpu.sync_copy(st0, aggp.at[cid, pl.ds(N + tid * AGG_CH, AGG_CH)])

  plsc.subcore_barrier()

  def gidx(chunk):
    return sblk.at[pl.ds(chunk * CH, CH)]

  for b in range(NBUF):
    pltpu.async_copy(f.at[gidx(b)], stages[b], sems[b])

  def echunk(c, carry):
    c3 = c * NBUF
    for b in range(NBUF):
      chunk = c3 + b
      pltpu.make_async_copy(f.at[gidx(chunk)], stages[b], sems[b]).wait()
      pltpu.sync_copy(stages[b], agg_acc.at[dblk.at[chunk]], add=True)
      nxt = chunk + NBUF

      @pl.when(nxt < NCHE)
      def _(b=b, nxt=nxt):
        pltpu.async_copy(f.at[gidx(nxt)], stages[b], sems[b])
    return carry

  lax.fori_loop(0, NCHE // NBUF, echunk, 0)
  for chunk in range(NCHE - NCHE % NBUF, NCHE):
    b = chunk % NBUF
    pltpu.make_async_copy(f.at[gidx(chunk)], stages[b], sems[b]).wait()
    pltpu.sync_copy(stages[b], agg_acc.at[dblk.at[chunk]], add=True)

  plsc.subcore_barrier()

  @pl.when(tid < NTEC - 1)
  def _():
    pltpu.sync_copy(agg_acc.at[pl.ds(tid * AGG_TILE, AGG_TILE)],
                    aggp.at[cid, pl.ds(tid * AGG_TILE, AGG_TILE)])

  @pl.when(tid == NTEC - 1)
  def _():
    pltpu.sync_copy(agg_acc.at[pl.ds(tid * AGG_TILE, 400)],
                    aggp.at[cid, pl.ds(tid * AGG_TILE, 400)])


_sc_edges = pl.kernel(
    _sc_edges_body,
    out_type=[jax.ShapeDtypeStruct((NSC, NP, D), jnp.float32)],
    mesh=_SC_MESH,
    scratch_types=[
        pltpu.VMEM_SHARED((AGG_ROWS, D), jnp.float32),
        pltpu.VMEM((EPT,), jnp.int32),
        pltpu.VMEM((NCHE, CH), jnp.int32),
        pltpu.VMEM((CH, D), jnp.float32),
        pltpu.VMEM((CH, D), jnp.float32),
        pltpu.SemaphoreType.DMA,
        pltpu.SemaphoreType.DMA,
    ],
    name="gin_sc_edges",
)


def _sc_readout_body(u0, u1, u2, h0, h1, h2, idsoff, ids1, ninf3, zhbm,
                     sumgp, maxgp,
                     sumg_acc, stage, stage2, ridx2, idsbuf, maxb, sem):
  del sem
  cid = lax.axis_index("c")
  tid = lax.axis_index("s")
  wid = cid * NTEC + tid

  pltpu.sync_copy(ninf3, maxb)
  pltpu.sync_copy(idsoff.at[wid], ridx2)
  pltpu.sync_copy(ids1.at[pl.ds(wid * NPT, NPT)], idsbuf.at[pl.ds(0, NPT)])

  @pl.when(tid < 11)
  def _():
    pltpu.sync_copy(zhbm.at[pl.ds(0, SG_CH)], stage)
    pltpu.sync_copy(stage, sumg_acc.at[pl.ds(tid * SG_CH, SG_CH)])

  plsc.subcore_barrier()

  nbase = wid * NPT
  for l, (u, h) in enumerate(((u0, h0), (u1, h1), (u2, h2))):
    for k in range(NRC):
      off = nbase + k * RC
      pltpu.sync_copy(u.at[pl.ds(off, RC)], stage.at[pl.ds(0, RC)])
      pltpu.sync_copy(stage.at[pl.ds(0, RC)],
                      sumg_acc.at[ridx2.at[l * NRC + k]], add=True)
      pltpu.sync_copy(h.at[pl.ds(off, RC)], stage2)

      def mbody(n, carry, k=k, l=l):
        g = idsbuf[pl.ds(k * RC + n, 16)][0] + l * BP
        for j in range(D // 16):
          sl = pl.ds(j * 16, 16)
          maxb[g, sl] = jnp.maximum(maxb[g, sl], stage2[n, sl])
        return carry

      lax.fori_loop(0, RC, mbody, 0)

  plsc.subcore_barrier()

  pltpu.sync_copy(maxb, maxgp.at[wid])

  @pl.when(tid < 11)
  def _():
    pltpu.sync_copy(sumg_acc.at[pl.ds(tid * SG_CH, SG_CH)],
                    sumgp.at[cid, pl.ds(tid * SG_CH, SG_CH)])


_sc_readout = pl.kernel(
    _sc_readout_body,
    out_type=[
        jax.ShapeDtypeStruct((NSC, BP3, D), jnp.float32),
        jax.ShapeDtypeStruct((NW, BP3, D), jnp.float32),
    ],
    mesh=_SC_MESH,
    scratch_types=[
        pltpu.VMEM_SHARED((BP3, D), jnp.float32),
        pltpu.VMEM((SG_CH, D), jnp.float32),
        pltpu.VMEM((RC, D), jnp.float32),
        pltpu.VMEM((NL * NRC, RC), jnp.int32),
        pltpu.VMEM((NPT + 16,), jnp.int32),
        pltpu.VMEM((BP3, D), jnp.float32),
        pltpu.SemaphoreType.DMA,
    ],
    name="gin_sc_readout",
)


def _row_spec(blk):
  return pl.BlockSpec((blk, D), lambda i: (i, 0))


def _full(shape):
  nd = len(shape)
  return pl.BlockSpec(shape, lambda i, nd=nd: (0,) * nd)


def _init_body(x_ref, wt_ref, bin_ref, g_ref, bt_ref, wg_ref, bg_ref,
               h_ref, f_ref, u_ref):
  x = x_ref[...]
  h = jnp.dot(x, wt_ref[...], preferred_element_type=jnp.float32)
  h = h + bin_ref[...]
  h_ref[...] = h
  f_ref[...] = h * g_ref[...] + bt_ref[...]
  wl = jnp.sum(h * wg_ref[...], axis=1, keepdims=True) + bg_ref[...]
  u_ref[...] = jax.nn.sigmoid(wl) * h


def _gru_body(f_ref, a_ref, h_ref, wa_ref, wb_ref, wh_ref, bih_ref,
              bhh_ref, g_ref, bt_ref, wg_ref, bg_ref,
              hn_ref, fn_ref, un_ref):
  f = f_ref[...]
  agg = a_ref[0] + a_ref[1]
  h = h_ref[...]
  fs = f * (1.0 + EPS_GIN)
  gi = jnp.dot(fs, wa_ref[...], preferred_element_type=jnp.float32)
  gi = gi + jnp.dot(agg, wb_ref[...], preferred_element_type=jnp.float32)
  gi = gi + bih_ref[...]
  gh = jnp.dot(f, wh_ref[...], preferred_element_type=jnp.float32)
  gh = gh + bhh_ref[...]
  ir, iz, inn = gi[:, :D], gi[:, D:2 * D], gi[:, 2 * D:]
  hr, hz, hn = gh[:, :D], gh[:, D:2 * D], gh[:, 2 * D:]
  r = jax.nn.sigmoid(ir + hr)
  z = jax.nn.sigmoid(iz + hz)
  n = jnp.tanh(inn + r * hn)
  new = (1.0 - z) * n + z * f
  new = jnp.where(new > 0, new, jnp.exp(jnp.minimum(new, 0.0)) - 1.0)
  hn2 = new + h
  hn_ref[...] = hn2
  fn_ref[...] = hn2 * g_ref[...] + bt_ref[...]
  wl = jnp.sum(hn2 * wg_ref[...], axis=1, keepdims=True) + bg_ref[...]
  un_ref[...] = jax.nn.sigmoid(wl) * hn2


def _r1_body(sp_ref, mp_ref, out_ref):
  s = sp_ref[0, 0] + sp_ref[1, 0]
  m = jnp.max(mp_ref[:, 0], axis=0)
  m = jnp.where(jnp.isneginf(m), 0.0, m)
  out_ref[0] = jnp.concatenate([s[:B], m[:B]], axis=-1)


def _r2_body(seq_ref, wif_ref, bif_ref, wib_ref, whb_ref, bib_ref,
             bhb_ref, bhf_ref, wo_ref, bo_ref, out_ref):
  x0 = seq_ref[0]
  g = jnp.dot(x0, wif_ref[...], preferred_element_type=jnp.float32)
  g = g + bif_ref[...] + bhf_ref[...]
  i, fgate, gg, o = (g[:, :D], g[:, D:2 * D], g[:, 2 * D:3 * D],
                     g[:, 3 * D:])
  c = jax.nn.sigmoid(i) * jnp.tanh(gg)
  hf0 = jax.nn.sigmoid(o) * jnp.tanh(c)
  hb = jnp.zeros((B, D), jnp.float32)
  cb = jnp.zeros((B, D), jnp.float32)
  for t in (2, 1, 0):
    x = seq_ref[t]
    g = jnp.dot(x, wib_ref[...], preferred_element_type=jnp.float32)
    g = g + bib_ref[...]
    g = g + jnp.dot(hb, whb_ref[...], preferred_element_type=jnp.float32)
    g = g + bhb_ref[...]
    i, fgate, gg, o = (g[:, :D], g[:, D:2 * D], g[:, 2 * D:3 * D],
                       g[:, 3 * D:])
    cb = jax.nn.sigmoid(fgate) * cb + jax.nn.sigmoid(i) * jnp.tanh(gg)
    hb = jax.nn.sigmoid(o) * jnp.tanh(cb)
  out = jnp.concatenate([hf0, hb], axis=-1)
  out = jnp.dot(out, wo_ref[...], preferred_element_type=jnp.float32)
  out_ref[...] = out + bo_ref[...]


def kernel(feats, edge_index, node_graph_ids, W_in, b_in, W_ih, W_hh,
           b_ih, b_hh, bn_gamma, bn_beta, Wg, bg, W_if, W_hf, b_if,
           b_hf, W_ib, W_hb, b_ib, b_hb, Wo, bo):
  f32 = jnp.float32
  inv = 1.0 / jnp.sqrt(jnp.asarray(1.0 + BN_EPS, f32))

  feats_p = jnp.pad(feats, ((0, NP - N), (0, 0)))
  ids_p = jnp.pad(node_graph_ids, (0, NP - N), constant_values=B)
  src1 = edge_index[0]
  dst2 = edge_index[1].reshape(NW, NCHE, CH)
  idsoff = (ids_p[None, :] +
            (jnp.arange(NL, dtype=jnp.int32) * BP)[:, None])
  idsoff = idsoff.reshape(NL, NW, NRC * RC).transpose(1, 0, 2)
  idsoff = idsoff.reshape(NW, NL * NRC, RC)
  zeros128 = jnp.zeros((CH, D), f32)
  ninf3 = jnp.full((BP3, D), -jnp.inf, f32)

  wt_in = W_in.T
  b_in2 = b_in.reshape(1, D)
  wa = W_ih[:, :D].T
  wb = W_ih[:, D:].T
  wh = W_hh.T
  bih = b_ih.reshape(1, 3 * D)
  bhh = b_hh.reshape(1, 3 * D)
  gam = (bn_gamma * inv).astype(f32)
  bet = bn_beta
  wg_row = Wg.reshape(1, D)
  bg2 = bg.reshape(1, 1)

  wif = W_if.T
  bif = b_if.reshape(1, 4 * D)
  bhf = b_hf.reshape(1, 4 * D)
  wib = W_ib.T
  whb = W_hb.T
  bib = b_ib.reshape(1, 4 * D)
  bhb = b_hb.reshape(1, 4 * D)
  wo_p = jnp.zeros((2 * D, 128), f32).at[:, :NC_OUT].set(Wo.T)
  bo_p = jnp.zeros((1, 128), f32).at[0, :NC_OUT].set(bo)

  init_call = pl.pallas_call(
      _init_body,
      grid=(NP // BLK,),
      in_specs=[
          _row_spec(BLK), _full((D, D)), _full((1, D)), _full((1, D)),
          _full((1, D)), _full((1, D)), _full((1, 1)),
      ],
      out_specs=[_row_spec(BLK)] * 3,
      out_shape=[jax.ShapeDtypeStruct((NP, D), f32)] * 3,
  )
  h0, f0, u0 = init_call(feats_p, wt_in, b_in2, gam[0].reshape(1, D),
                         bet[0].reshape(1, D), wg_row, bg2)

  gru_call = pl.pallas_call(
      _gru_body,
      grid=(NP // BLK,),
      in_specs=[
          _row_spec(BLK),
          pl.BlockSpec((NSC, BLK, D), lambda i: (0, i, 0)),
          _row_spec(BLK),
          _full((D, 3 * D)), _full((D, 3 * D)), _full((D, 3 * D)),
          _full((1, 3 * D)), _full((1, 3 * D)),
          _full((1, D)), _full((1, D)), _full((1, D)), _full((1, 1)),
      ],
      out_specs=[_row_spec(BLK)] * 3,
      out_shape=[jax.ShapeDtypeStruct((NP, D), f32)] * 3,
  )

  (aggp0,) = _sc_edges(f0, src1, dst2, zeros128)
  h1, f1, u1 = gru_call(f0, aggp0, h0, wa, wb, wh, bih, bhh,
                        gam[1].reshape(1, D), bet[1].reshape(1, D),
                        wg_row, bg2)
  (aggp1,) = _sc_edges(f1, src1, dst2, zeros128)
  h2, _, u2 = gru_call(f1, aggp1, h1, wa, wb, wh, bih, bhh,
                       gam[2].reshape(1, D), bet[2].reshape(1, D),
                       wg_row, bg2)

  sgp, mgp = _sc_readout(u0, u1, u2, h0, h1, h2, idsoff, ids_p, ninf3,
                         zeros128)
  sgp = sgp.reshape(NSC, NL, BP, D)
  mgp = mgp.reshape(NW, NL, BP, D)

  r1_call = pl.pallas_call(
      _r1_body,
      grid=(NL,),
      in_specs=[
          pl.BlockSpec((NSC, 1, BP, D), lambda i: (0, i, 0, 0)),
          pl.BlockSpec((NW, 1, BP, D), lambda i: (0, i, 0, 0)),
      ],
      out_specs=pl.BlockSpec((1, B, 2 * D), lambda i: (i, 0, 0)),
      out_shape=jax.ShapeDtypeStruct((NL, B, 2 * D), f32),
  )
  seq = r1_call(sgp, mgp)

  r2_call = pl.pallas_call(
      _r2_body,
      grid=(1,),
      in_specs=[
          _full((NL, B, 2 * D)),
          _full((2 * D, 4 * D)), _full((1, 4 * D)),
          _full((2 * D, 4 * D)), _full((D, 4 * D)), _full((1, 4 * D)),
          _full((1, 4 * D)), _full((1, 4 * D)),
          _full((2 * D, 128)), _full((1, 128)),
      ],
      out_specs=_full((B, 128)),
      out_shape=jax.ShapeDtypeStruct((B, 128), f32),
  )
  out = r2_call(seq, wif, bif, wib, whb, bib, bhb, bhf, wo_p, bo_p)
  return out[:, :NC_OUT]


# speedup vs baseline: 7.8327x; 1.0384x over previous
import functools

import jax
import jax.numpy as jnp
from jax import lax
from jax.experimental import pallas as pl
from jax.experimental.pallas import tpu as pltpu
from jax.experimental.pallas import tpu_sc as plsc

N = 10000
E = 320000
D = 128
B = 256
NC_OUT = 12
EPS_GIN = 1e-05
BN_EPS = 1e-05

NSC = 2
NTEC = 16
NW = NSC * NTEC

NPT = 320
NP = NW * NPT
RC = 64
NRC = NPT // RC

CH = 80
EPT = E // NW
NCHE = EPT // CH
NBUF = 2

BP = 264
NL = 3
BP3 = NL * BP

AGG_ROWS = N + 8
AGG_TILE = 640
AGG_CH = 80
SG_CH = 72

BLK = 1024

_SC_MESH = plsc.VectorSubcoreMesh(
    core_axis_name="c", subcore_axis_name="s", num_cores=NSC,
    num_subcores=NTEC)


def _sc_edges_body(f, src1, dst2, zhbm, aggp,
                   agg_acc, sblk, dblk, st0, st1, sem0, sem1):
  cid = lax.axis_index("c")
  tid = lax.axis_index("s")
  wid = cid * NTEC + tid
  stages = (st0, st1)
  sems = (sem0, sem1)

  pltpu.sync_copy(src1.at[pl.ds(wid * EPT, EPT)], sblk)
  pltpu.sync_copy(dst2.at[wid], dblk)
  pltpu.sync_copy(zhbm, st0)
  nch = jnp.where(tid < NTEC - 1, 8, 5)

  def zbody(k, carry):
    pltpu.sync_copy(st0.at[pl.ds(0, AGG_CH)],
                    agg_acc.at[pl.ds(tid * AGG_TILE + k * AGG_CH, AGG_CH)])
    return carry

  lax.fori_loop(0, nch, zbody, 0)

  @pl.when(tid < 3)
  def _():
    pltpu.sync_copy(st0, aggp.at[cid, pl.ds(N + tid * AGG_CH, AGG_CH)])

  plsc.subcore_barrier()

  def gidx(chunk):
    return sblk.at[pl.ds(chunk * CH, CH)]

  for b in range(NBUF):
    pltpu.async_copy(f.at[gidx(b)], stages[b], sems[b])

  def echunk(c, carry):
    c3 = c * NBUF
    for b in range(NBUF):
      chunk = c3 + b
      pltpu.make_async_copy(f.at[gidx(chunk)], stages[b], sems[b]).wait()
      pltpu.sync_copy(stages[b], agg_acc.at[dblk.at[chunk]], add=True)
      nxt = chunk + NBUF

      @pl.when(nxt < NCHE)
      def _(b=b, nxt=nxt):
        pltpu.async_copy(f.at[gidx(nxt)], stages[b], sems[b])
    return carry

  lax.fori_loop(0, NCHE // NBUF, echunk, 0)
  for chunk in range(NCHE - NCHE % NBUF, NCHE):
    b = chunk % NBUF
    pltpu.make_async_copy(f.at[gidx(chunk)], stages[b], sems[b]).wait()
    pltpu.sync_copy(stages[b], agg_acc.at[dblk.at[chunk]], add=True)

  plsc.subcore_barrier()

  @pl.when(tid < NTEC - 1)
  def _():
    pltpu.sync_copy(agg_acc.at[pl.ds(tid * AGG_TILE, AGG_TILE)],
                    aggp.at[cid, pl.ds(tid * AGG_TILE, AGG_TILE)])

  @pl.when(tid == NTEC - 1)
  def _():
    pltpu.sync_copy(agg_acc.at[pl.ds(tid * AGG_TILE, 400)],
                    aggp.at[cid, pl.ds(tid * AGG_TILE, 400)])


_sc_edges = pl.kernel(
    _sc_edges_body,
    out_type=[jax.ShapeDtypeStruct((NSC, NP, D), jnp.float32)],
    mesh=_SC_MESH,
    scratch_types=[
        pltpu.VMEM_SHARED((AGG_ROWS, D), jnp.float32),
        pltpu.VMEM((EPT,), jnp.int32),
        pltpu.VMEM((NCHE, CH), jnp.int32),
        pltpu.VMEM((CH, D), jnp.float32),
        pltpu.VMEM((CH, D), jnp.float32),
        pltpu.SemaphoreType.DMA,
        pltpu.SemaphoreType.DMA,
    ],
    name="gin_sc_edges",
)


def _sc_readout_body(u0, u1, u2, h0, h1, h2, idsoff, ids1, ninf3, zhbm,
                     sumgp, maxgp,
                     sumg_acc, stage, stage2, ridx2, idsbuf, maxb, sem):
  del sem
  cid = lax.axis_index("c")
  tid = lax.axis_index("s")
  wid = cid * NTEC + tid

  pltpu.sync_copy(ninf3, maxb)
  pltpu.sync_copy(idsoff.at[wid], ridx2)
  pltpu.sync_copy(ids1.at[pl.ds(wid * NPT, NPT)], idsbuf.at[pl.ds(0, NPT)])

  @pl.when(tid < 11)
  def _():
    pltpu.sync_copy(zhbm.at[pl.ds(0, SG_CH)], stage)
    pltpu.sync_copy(stage, sumg_acc.at[pl.ds(tid * SG_CH, SG_CH)])

  plsc.subcore_barrier()

  nbase = wid * NPT
  ninf16 = jnp.full((16,), -jnp.inf, jnp.float32)
  g0 = idsbuf[pl.ds(0, 16)][0]
  for l, (u, h) in enumerate(((u0, h0), (u1, h1), (u2, h2))):
    carry = (g0,) + (ninf16,) * (D // 16)
    for k in range(NRC):
      off = nbase + k * RC
      pltpu.sync_copy(u.at[pl.ds(off, RC)], stage.at[pl.ds(0, RC)])
      pltpu.sync_copy(stage.at[pl.ds(0, RC)],
                      sumg_acc.at[ridx2.at[l * NRC + k]], add=True)
      pltpu.sync_copy(h.at[pl.ds(off, RC)], stage2)

      def mbody(nb, carry, k=k, l=l):
        ids16 = idsbuf[pl.ds(k * RC + nb * 16, 16)]
        for i in range(16):
          gprev = carry[0]
          m = carry[1:]
          g = ids16[i]
          same = g == gprev

          @pl.when(jnp.logical_not(same))
          def _(gprev=gprev, m=m, l=l):
            for j in range(D // 16):
              maxb[pl.ds((gprev + l * BP) * D + j * 16, 16)] = m[j]

          n = nb * 16 + i
          carry = (g,) + tuple(
              jnp.where(same, jnp.maximum(m[j],
                                          stage2[n, pl.ds(j * 16, 16)]),
                        stage2[n, pl.ds(j * 16, 16)])
              for j in range(D // 16))
        return carry

      carry = lax.fori_loop(0, RC // 16, mbody, carry)
    for j in range(D // 16):
      maxb[pl.ds((carry[0] + l * BP) * D + j * 16, 16)] = carry[1 + j]

  plsc.subcore_barrier()

  pltpu.sync_copy(maxb, maxgp.at[wid])

  @pl.when(tid < 11)
  def _():
    pltpu.sync_copy(sumg_acc.at[pl.ds(tid * SG_CH, SG_CH)],
                    sumgp.at[cid, pl.ds(tid * SG_CH, SG_CH)])


_sc_readout = pl.kernel(
    _sc_readout_body,
    out_type=[
        jax.ShapeDtypeStruct((NSC, BP3, D), jnp.float32),
        jax.ShapeDtypeStruct((NW, BP3 * D), jnp.float32),
    ],
    mesh=_SC_MESH,
    scratch_types=[
        pltpu.VMEM_SHARED((BP3, D), jnp.float32),
        pltpu.VMEM((SG_CH, D), jnp.float32),
        pltpu.VMEM((RC, D), jnp.float32),
        pltpu.VMEM((NL * NRC, RC), jnp.int32),
        pltpu.VMEM((NPT + 16,), jnp.int32),
        pltpu.VMEM((BP3 * D,), jnp.float32),
        pltpu.SemaphoreType.DMA,
    ],
    name="gin_sc_readout",
)


def _row_spec(blk):
  return pl.BlockSpec((blk, D), lambda i: (i, 0))


def _full(shape):
  nd = len(shape)
  return pl.BlockSpec(shape, lambda i, nd=nd: (0,) * nd)


def _init_body(x_ref, wt_ref, bin_ref, g_ref, bt_ref, wg_ref, bg_ref,
               h_ref, f_ref, u_ref):
  x = x_ref[...]
  h = jnp.dot(x, wt_ref[...], preferred_element_type=jnp.float32)
  h = h + bin_ref[...]
  h_ref[...] = h
  f_ref[...] = h * g_ref[...] + bt_ref[...]
  wl = jnp.sum(h * wg_ref[...], axis=1, keepdims=True) + bg_ref[...]
  u_ref[...] = jax.nn.sigmoid(wl) * h


def _gru_body(f_ref, a_ref, h_ref, wa_ref, wb_ref, wh_ref, bih_ref,
              bhh_ref, g_ref, bt_ref, wg_ref, bg_ref,
              hn_ref, fn_ref, un_ref):
  f = f_ref[...]
  agg = a_ref[0] + a_ref[1]
  h = h_ref[...]
  fs = f * (1.0 + EPS_GIN)
  gi = jnp.dot(fs, wa_ref[...], preferred_element_type=jnp.float32)
  gi = gi + jnp.dot(agg, wb_ref[...], preferred_element_type=jnp.float32)
  gi = gi + bih_ref[...]
  gh = jnp.dot(f, wh_ref[...], preferred_element_type=jnp.float32)
  gh = gh + bhh_ref[...]
  ir, iz, inn = gi[:, :D], gi[:, D:2 * D], gi[:, 2 * D:]
  hr, hz, hn = gh[:, :D], gh[:, D:2 * D], gh[:, 2 * D:]
  r = jax.nn.sigmoid(ir + hr)
  z = jax.nn.sigmoid(iz + hz)
  n = jnp.tanh(inn + r * hn)
  new = (1.0 - z) * n + z * f
  new = jnp.where(new > 0, new, jnp.exp(jnp.minimum(new, 0.0)) - 1.0)
  hn2 = new + h
  hn_ref[...] = hn2
  fn_ref[...] = hn2 * g_ref[...] + bt_ref[...]
  wl = jnp.sum(hn2 * wg_ref[...], axis=1, keepdims=True) + bg_ref[...]
  un_ref[...] = jax.nn.sigmoid(wl) * hn2


def _r1_body(sp_ref, mp_ref, out_ref):
  s = sp_ref[0, 0] + sp_ref[1, 0]
  m = jnp.max(mp_ref[:, 0], axis=0)
  m = jnp.where(jnp.isneginf(m), 0.0, m)
  out_ref[0] = jnp.concatenate([s[:B], m[:B]], axis=-1)


def _r2_body(seq_ref, wif_ref, bif_ref, wib_ref, whb_ref, bib_ref,
             bhb_ref, bhf_ref, wo_ref, bo_ref, out_ref):
  x0 = seq_ref[0]
  g = jnp.dot(x0, wif_ref[...], preferred_element_type=jnp.float32)
  g = g + bif_ref[...] + bhf_ref[...]
  i, fgate, gg, o = (g[:, :D], g[:, D:2 * D], g[:, 2 * D:3 * D],
                     g[:, 3 * D:])
  c = jax.nn.sigmoid(i) * jnp.tanh(gg)
  hf0 = jax.nn.sigmoid(o) * jnp.tanh(c)
  hb = jnp.zeros((B, D), jnp.float32)
  cb = jnp.zeros((B, D), jnp.float32)
  for t in (2, 1, 0):
    x = seq_ref[t]
    g = jnp.dot(x, wib_ref[...], preferred_element_type=jnp.float32)
    g = g + bib_ref[...]
    g = g + jnp.dot(hb, whb_ref[...], preferred_element_type=jnp.float32)
    g = g + bhb_ref[...]
    i, fgate, gg, o = (g[:, :D], g[:, D:2 * D], g[:, 2 * D:3 * D],
                       g[:, 3 * D:])
    cb = jax.nn.sigmoid(fgate) * cb + jax.nn.sigmoid(i) * jnp.tanh(gg)
    hb = jax.nn.sigmoid(o) * jnp.tanh(cb)
  out = jnp.concatenate([hf0, hb], axis=-1)
  out = jnp.dot(out, wo_ref[...], preferred_element_type=jnp.float32)
  out_ref[...] = out + bo_ref[...]


def kernel(feats, edge_index, node_graph_ids, W_in, b_in, W_ih, W_hh,
           b_ih, b_hh, bn_gamma, bn_beta, Wg, bg, W_if, W_hf, b_if,
           b_hf, W_ib, W_hb, b_ib, b_hb, Wo, bo):
  f32 = jnp.float32
  inv = 1.0 / jnp.sqrt(jnp.asarray(1.0 + BN_EPS, f32))

  feats_p = jnp.pad(feats, ((0, NP - N), (0, 0)))
  ids_p = jnp.pad(node_graph_ids, (0, NP - N), constant_values=B)
  src1 = edge_index[0]
  dst2 = edge_index[1].reshape(NW, NCHE, CH)
  idsoff = (ids_p[None, :] +
            (jnp.arange(NL, dtype=jnp.int32) * BP)[:, None])
  idsoff = idsoff.reshape(NL, NW, NRC * RC).transpose(1, 0, 2)
  idsoff = idsoff.reshape(NW, NL * NRC, RC)
  zeros128 = jnp.zeros((CH, D), f32)
  ninf3 = jnp.full((BP3 * D,), -jnp.inf, f32)

  wt_in = W_in.T
  b_in2 = b_in.reshape(1, D)
  wa = W_ih[:, :D].T
  wb = W_ih[:, D:].T
  wh = W_hh.T
  bih = b_ih.reshape(1, 3 * D)
  bhh = b_hh.reshape(1, 3 * D)
  gam = (bn_gamma * inv).astype(f32)
  bet = bn_beta
  wg_row = Wg.reshape(1, D)
  bg2 = bg.reshape(1, 1)

  wif = W_if.T
  bif = b_if.reshape(1, 4 * D)
  bhf = b_hf.reshape(1, 4 * D)
  wib = W_ib.T
  whb = W_hb.T
  bib = b_ib.reshape(1, 4 * D)
  bhb = b_hb.reshape(1, 4 * D)
  wo_p = jnp.zeros((2 * D, 128), f32).at[:, :NC_OUT].set(Wo.T)
  bo_p = jnp.zeros((1, 128), f32).at[0, :NC_OUT].set(bo)

  init_call = pl.pallas_call(
      _init_body,
      grid=(NP // BLK,),
      in_specs=[
          _row_spec(BLK), _full((D, D)), _full((1, D)), _full((1, D)),
          _full((1, D)), _full((1, D)), _full((1, 1)),
      ],
      out_specs=[_row_spec(BLK)] * 3,
      out_shape=[jax.ShapeDtypeStruct((NP, D), f32)] * 3,
  )
  h0, f0, u0 = init_call(feats_p, wt_in, b_in2, gam[0].reshape(1, D),
                         bet[0].reshape(1, D), wg_row, bg2)

  gru_call = pl.pallas_call(
      _gru_body,
      grid=(NP // BLK,),
      in_specs=[
          _row_spec(BLK),
          pl.BlockSpec((NSC, BLK, D), lambda i: (0, i, 0)),
          _row_spec(BLK),
          _full((D, 3 * D)), _full((D, 3 * D)), _full((D, 3 * D)),
          _full((1, 3 * D)), _full((1, 3 * D)),
          _full((1, D)), _full((1, D)), _full((1, D)), _full((1, 1)),
      ],
      out_specs=[_row_spec(BLK)] * 3,
      out_shape=[jax.ShapeDtypeStruct((NP, D), f32)] * 3,
  )

  (aggp0,) = _sc_edges(f0, src1, dst2, zeros128)
  h1, f1, u1 = gru_call(f0, aggp0, h0, wa, wb, wh, bih, bhh,
                        gam[1].reshape(1, D), bet[1].reshape(1, D),
                        wg_row, bg2)
  (aggp1,) = _sc_edges(f1, src1, dst2, zeros128)
  h2, _, u2 = gru_call(f1, aggp1, h1, wa, wb, wh, bih, bhh,
                       gam[2].reshape(1, D), bet[2].reshape(1, D),
                       wg_row, bg2)

  sgp, mgp = _sc_readout(u0, u1, u2, h0, h1, h2, idsoff, ids_p, ninf3,
                         zeros128)
  sgp = sgp.reshape(NSC, NL, BP, D)
  mgp = mgp.reshape(NW, NL, BP, D)

  r1_call = pl.pallas_call(
      _r1_body,
      grid=(NL,),
      in_specs=[
          pl.BlockSpec((NSC, 1, BP, D), lambda i: (0, i, 0, 0)),
          pl.BlockSpec((NW, 1, BP, D), lambda i: (0, i, 0, 0)),
      ],
      out_specs=pl.BlockSpec((1, B, 2 * D), lambda i: (i, 0, 0)),
      out_shape=jax.ShapeDtypeStruct((NL, B, 2 * D), f32),
  )
  seq = r1_call(sgp, mgp)

  r2_call = pl.pallas_call(
      _r2_body,
      grid=(1,),
      in_specs=[
          _full((NL, B, 2 * D)),
          _full((2 * D, 4 * D)), _full((1, 4 * D)),
          _full((2 * D, 4 * D)), _full((D, 4 * D)), _full((1, 4 * D)),
          _full((1, 4 * D)), _full((1, 4 * D)),
          _full((2 * D, 128)), _full((1, 128)),
      ],
      out_specs=_full((B, 128)),
      out_shape=jax.ShapeDtypeStruct((B, 128), f32),
  )
  out = r2_call(seq, wif, bif, wib, whb, bib, bhb, bhf, wo_p, bo_p)
  return out[:, :NC_OUT]


# speedup vs baseline: 8.0863x; 1.0324x over previous
import functools

import jax
import jax.numpy as jnp
from jax import lax
from jax.experimental import pallas as pl
from jax.experimental.pallas import tpu as pltpu
from jax.experimental.pallas import tpu_sc as plsc

N = 10000
E = 320000
D = 128
B = 256
NC_OUT = 12
EPS_GIN = 1e-05
BN_EPS = 1e-05

NSC = 2
NTEC = 16
NW = NSC * NTEC

NPT = 320
NP = NW * NPT
RC = 64
NRC = NPT // RC

CH = 80
EPT = E // NW
NCHE = EPT // CH
NBUF = 2

BP = 264
NL = 3
BP3 = NL * BP

AGG_ROWS = N + 8
AGG_TILE = 640
AGG_CH = 80
SG_CH = 72

BLK = 1024

_SC_MESH = plsc.VectorSubcoreMesh(
    core_axis_name="c", subcore_axis_name="s", num_cores=NSC,
    num_subcores=NTEC)


def _sc_edges_body(f, src1, dst2, zhbm, aggp,
                   agg_acc, sblk, dblk, st0, st1, sem0, sem1):
  cid = lax.axis_index("c")
  tid = lax.axis_index("s")
  wid = cid * NTEC + tid
  stages = (st0, st1)
  sems = (sem0, sem1)

  pltpu.sync_copy(src1.at[pl.ds(wid * EPT, EPT)], sblk)
  pltpu.sync_copy(dst2.at[wid], dblk)
  pltpu.sync_copy(zhbm, st0)
  nch = jnp.where(tid < NTEC - 1, 8, 5)

  def zbody(k, carry):
    pltpu.sync_copy(st0.at[pl.ds(0, AGG_CH)],
                    agg_acc.at[pl.ds(tid * AGG_TILE + k * AGG_CH, AGG_CH)])
    return carry

  lax.fori_loop(0, nch, zbody, 0)

  @pl.when(tid < 3)
  def _():
    pltpu.sync_copy(st0, aggp.at[cid, pl.ds(N + tid * AGG_CH, AGG_CH)])

  plsc.subcore_barrier()

  def gidx(chunk):
    return sblk.at[pl.ds(chunk * CH, CH)]

  for b in range(NBUF):
    pltpu.async_copy(f.at[gidx(b)], stages[b], sems[b])

  def echunk(c, carry):
    c3 = c * NBUF
    for b in range(NBUF):
      chunk = c3 + b
      pltpu.make_async_copy(f.at[gidx(chunk)], stages[b], sems[b]).wait()
      pltpu.sync_copy(stages[b], agg_acc.at[dblk.at[chunk]], add=True)
      nxt = chunk + NBUF

      @pl.when(nxt < NCHE)
      def _(b=b, nxt=nxt):
        pltpu.async_copy(f.at[gidx(nxt)], stages[b], sems[b])
    return carry

  lax.fori_loop(0, NCHE // NBUF, echunk, 0)
  for chunk in range(NCHE - NCHE % NBUF, NCHE):
    b = chunk % NBUF
    pltpu.make_async_copy(f.at[gidx(chunk)], stages[b], sems[b]).wait()
    pltpu.sync_copy(stages[b], agg_acc.at[dblk.at[chunk]], add=True)

  plsc.subcore_barrier()

  @pl.when(tid < NTEC - 1)
  def _():
    pltpu.sync_copy(agg_acc.at[pl.ds(tid * AGG_TILE, AGG_TILE)],
                    aggp.at[cid, pl.ds(tid * AGG_TILE, AGG_TILE)])

  @pl.when(tid == NTEC - 1)
  def _():
    pltpu.sync_copy(agg_acc.at[pl.ds(tid * AGG_TILE, 400)],
                    aggp.at[cid, pl.ds(tid * AGG_TILE, 400)])


_sc_edges = pl.kernel(
    _sc_edges_body,
    out_type=[jax.ShapeDtypeStruct((NSC, NP, D), jnp.float32)],
    mesh=_SC_MESH,
    scratch_types=[
        pltpu.VMEM_SHARED((AGG_ROWS, D), jnp.float32),
        pltpu.VMEM((EPT,), jnp.int32),
        pltpu.VMEM((NCHE, CH), jnp.int32),
        pltpu.VMEM((CH, D), jnp.float32),
        pltpu.VMEM((CH, D), jnp.float32),
        pltpu.SemaphoreType.DMA,
        pltpu.SemaphoreType.DMA,
    ],
    name="gin_sc_edges",
)


def _sc_readout_body(u0, u1, u2, h0, h1, h2, idsoff, ids1, ninf3, zhbm,
                     sumgp, maxgp,
                     sumg_acc, stage, stage2, ridx2, idsbuf, maxb, sem):
  del sem
  cid = lax.axis_index("c")
  tid = lax.axis_index("s")
  wid = cid * NTEC + tid

  pltpu.sync_copy(ninf3, maxb)
  for l in range(NL):
    pltpu.sync_copy(idsoff.at[l, wid], ridx2.at[pl.ds(l * NRC, NRC)])
  pltpu.sync_copy(ids1.at[pl.ds(wid * NPT, NPT)], idsbuf.at[pl.ds(0, NPT)])

  @pl.when(tid < 11)
  def _():
    pltpu.sync_copy(zhbm.at[pl.ds(0, SG_CH)], stage)
    pltpu.sync_copy(stage, sumg_acc.at[pl.ds(tid * SG_CH, SG_CH)])

  plsc.subcore_barrier()

  nbase = wid * NPT
  ninf16 = jnp.full((16,), -jnp.inf, jnp.float32)
  g0 = idsbuf[pl.ds(0, 16)][0]
  for l, (u, h) in enumerate(((u0, h0), (u1, h1), (u2, h2))):
    carry = (g0,) + (ninf16,) * (D // 16)
    for k in range(NRC):
      off = nbase + k * RC
      pltpu.sync_copy(u.at[pl.ds(off, RC)], stage.at[pl.ds(0, RC)])
      pltpu.sync_copy(stage.at[pl.ds(0, RC)],
                      sumg_acc.at[ridx2.at[l * NRC + k]], add=True)
      pltpu.sync_copy(h.at[pl.ds(off, RC)], stage2)

      def mbody(nb, carry, k=k, l=l):
        ids16 = idsbuf[pl.ds(k * RC + nb * 16, 16)]
        for i in range(16):
          gprev = carry[0]
          m = carry[1:]
          g = ids16[i]
          same = g == gprev

          @pl.when(jnp.logical_not(same))
          def _(gprev=gprev, m=m, l=l):
            r = gprev + l * BP
            for j in range(D // 16):
              sl = pl.ds(j * 16, 16)
              maxb[r, sl] = jnp.maximum(maxb[r, sl], m[j])

          n = nb * 16 + i
          carry = (g,) + tuple(
              jnp.where(same, jnp.maximum(m[j],
                                          stage2[n, pl.ds(j * 16, 16)]),
                        stage2[n, pl.ds(j * 16, 16)])
              for j in range(D // 16))
        return carry

      carry = lax.fori_loop(0, RC // 16, mbody, carry)
    r = carry[0] + l * BP
    for j in range(D // 16):
      sl = pl.ds(j * 16, 16)
      maxb[r, sl] = jnp.maximum(maxb[r, sl], carry[1 + j])

  plsc.subcore_barrier()

  pltpu.sync_copy(maxb, maxgp.at[wid])

  @pl.when(tid < 11)
  def _():
    pltpu.sync_copy(sumg_acc.at[pl.ds(tid * SG_CH, SG_CH)],
                    sumgp.at[cid, pl.ds(tid * SG_CH, SG_CH)])


_sc_readout = pl.kernel(
    _sc_readout_body,
    out_type=[
        jax.ShapeDtypeStruct((NSC, BP3, D), jnp.float32),
        jax.ShapeDtypeStruct((NW, BP3, D), jnp.float32),
    ],
    mesh=_SC_MESH,
    scratch_types=[
        pltpu.VMEM_SHARED((BP3, D), jnp.float32),
        pltpu.VMEM((SG_CH, D), jnp.float32),
        pltpu.VMEM((RC, D), jnp.float32),
        pltpu.VMEM((NL * NRC, RC), jnp.int32),
        pltpu.VMEM((NPT + 16,), jnp.int32),
        pltpu.VMEM((BP3, D), jnp.float32),
        pltpu.SemaphoreType.DMA,
    ],
    name="gin_sc_readout",
)


def _row_spec(blk):
  return pl.BlockSpec((blk, D), lambda i: (i, 0))


def _full(shape):
  nd = len(shape)
  return pl.BlockSpec(shape, lambda i, nd=nd: (0,) * nd)


def _init_body(x_ref, wt_ref, bin_ref, g_ref, bt_ref, wg_ref, bg_ref,
               h_ref, f_ref, u_ref):
  x = x_ref[...]
  h = jnp.dot(x, wt_ref[...], preferred_element_type=jnp.float32)
  h = h + bin_ref[...]
  h_ref[...] = h
  f_ref[...] = h * g_ref[...] + bt_ref[...]
  wl = jnp.sum(h * wg_ref[...], axis=1, keepdims=True) + bg_ref[...]
  u_ref[...] = jax.nn.sigmoid(wl) * h


def _gru_body(f_ref, a_ref, h_ref, wa_ref, wb_ref, wh_ref, bih_ref,
              bhh_ref, g_ref, bt_ref, wg_ref, bg_ref,
              hn_ref, fn_ref, un_ref):
  f = f_ref[...]
  agg = a_ref[0] + a_ref[1]
  h = h_ref[...]
  fs = f * (1.0 + EPS_GIN)
  gi = jnp.dot(fs, wa_ref[...], preferred_element_type=jnp.float32)
  gi = gi + jnp.dot(agg, wb_ref[...], preferred_element_type=jnp.float32)
  gi = gi + bih_ref[...]
  gh = jnp.dot(f, wh_ref[...], preferred_element_type=jnp.float32)
  gh = gh + bhh_ref[...]
  ir, iz, inn = gi[:, :D], gi[:, D:2 * D], gi[:, 2 * D:]
  hr, hz, hn = gh[:, :D], gh[:, D:2 * D], gh[:, 2 * D:]
  r = jax.nn.sigmoid(ir + hr)
  z = jax.nn.sigmoid(iz + hz)
  n = jnp.tanh(inn + r * hn)
  new = (1.0 - z) * n + z * f
  new = jnp.where(new > 0, new, jnp.exp(jnp.minimum(new, 0.0)) - 1.0)
  hn2 = new + h
  hn_ref[...] = hn2
  fn_ref[...] = hn2 * g_ref[...] + bt_ref[...]
  wl = jnp.sum(hn2 * wg_ref[...], axis=1, keepdims=True) + bg_ref[...]
  un_ref[...] = jax.nn.sigmoid(wl) * hn2


def _gru2_body(f_ref, a_ref, h_ref, wa_ref, wb_ref, wh_ref, bih_ref,
               bhh_ref, wg_ref, bg_ref, hn_ref, un_ref):
  f = f_ref[...]
  agg = a_ref[0] + a_ref[1]
  h = h_ref[...]
  fs = f * (1.0 + EPS_GIN)
  gi = jnp.dot(fs, wa_ref[...], preferred_element_type=jnp.float32)
  gi = gi + jnp.dot(agg, wb_ref[...], preferred_element_type=jnp.float32)
  gi = gi + bih_ref[...]
  gh = jnp.dot(f, wh_ref[...], preferred_element_type=jnp.float32)
  gh = gh + bhh_ref[...]
  ir, iz, inn = gi[:, :D], gi[:, D:2 * D], gi[:, 2 * D:]
  hr, hz, hn = gh[:, :D], gh[:, D:2 * D], gh[:, 2 * D:]
  r = jax.nn.sigmoid(ir + hr)
  z = jax.nn.sigmoid(iz + hz)
  n = jnp.tanh(inn + r * hn)
  new = (1.0 - z) * n + z * f
  new = jnp.where(new > 0, new, jnp.exp(jnp.minimum(new, 0.0)) - 1.0)
  hn2 = new + h
  hn_ref[...] = hn2
  wl = jnp.sum(hn2 * wg_ref[...], axis=1, keepdims=True) + bg_ref[...]
  un_ref[...] = jax.nn.sigmoid(wl) * hn2


def _r1_body(sp_ref, mp_ref, out_ref):
  s = sp_ref[0, 0] + sp_ref[1, 0]
  m = jnp.max(mp_ref[:, 0], axis=0)
  m = jnp.where(jnp.isneginf(m), 0.0, m)
  out_ref[0] = jnp.concatenate([s[:B], m[:B]], axis=-1)


def _r2_body(seq_ref, wif_ref, bif_ref, wib_ref, whb_ref, bib_ref,
             bhb_ref, bhf_ref, wo_ref, bo_ref, out_ref):
  x0 = seq_ref[0]
  g = jnp.dot(x0, wif_ref[...], preferred_element_type=jnp.float32)
  g = g + bif_ref[...] + bhf_ref[...]
  i, fgate, gg, o = (g[:, :D], g[:, D:2 * D], g[:, 2 * D:3 * D],
                     g[:, 3 * D:])
  c = jax.nn.sigmoid(i) * jnp.tanh(gg)
  hf0 = jax.nn.sigmoid(o) * jnp.tanh(c)
  hb = jnp.zeros((B, D), jnp.float32)
  cb = jnp.zeros((B, D), jnp.float32)
  for t in (2, 1, 0):
    x = seq_ref[t]
    g = jnp.dot(x, wib_ref[...], preferred_element_type=jnp.float32)
    g = g + bib_ref[...]
    g = g + jnp.dot(hb, whb_ref[...], preferred_element_type=jnp.float32)
    g = g + bhb_ref[...]
    i, fgate, gg, o = (g[:, :D], g[:, D:2 * D], g[:, 2 * D:3 * D],
                       g[:, 3 * D:])
    cb = jax.nn.sigmoid(fgate) * cb + jax.nn.sigmoid(i) * jnp.tanh(gg)
    hb = jax.nn.sigmoid(o) * jnp.tanh(cb)
  out = jnp.concatenate([hf0, hb], axis=-1)
  out = jnp.dot(out, wo_ref[...], preferred_element_type=jnp.float32)
  out_ref[...] = out + bo_ref[...]


def kernel(feats, edge_index, node_graph_ids, W_in, b_in, W_ih, W_hh,
           b_ih, b_hh, bn_gamma, bn_beta, Wg, bg, W_if, W_hf, b_if,
           b_hf, W_ib, W_hb, b_ib, b_hb, Wo, bo):
  f32 = jnp.float32
  inv = 1.0 / jnp.sqrt(jnp.asarray(1.0 + BN_EPS, f32))

  ids_p = jnp.pad(node_graph_ids, (0, NP - N), constant_values=B)
  src1 = edge_index[0]
  dst2 = edge_index[1].reshape(NW, NCHE, CH)
  idsoff = (ids_p[None, :] +
            (jnp.arange(NL, dtype=jnp.int32) * BP)[:, None])
  idsoff = idsoff.reshape(NL, NW, NRC, RC)
  zeros128 = jnp.zeros((CH, D), f32)
  ninf3 = jnp.full((BP3, D), -jnp.inf, f32)

  wt_in = W_in.T
  b_in2 = b_in.reshape(1, D)
  wa = W_ih[:, :D].T
  wb = W_ih[:, D:].T
  wh = W_hh.T
  bih = b_ih.reshape(1, 3 * D)
  bhh = b_hh.reshape(1, 3 * D)
  gam = (bn_gamma * inv).astype(f32)
  bet = bn_beta
  wg_row = Wg.reshape(1, D)
  bg2 = bg.reshape(1, 1)

  wif = W_if.T
  bif = b_if.reshape(1, 4 * D)
  bhf = b_hf.reshape(1, 4 * D)
  wib = W_ib.T
  whb = W_hb.T
  bib = b_ib.reshape(1, 4 * D)
  bhb = b_hb.reshape(1, 4 * D)
  wo_p = jnp.zeros((2 * D, 128), f32).at[:, :NC_OUT].set(Wo.T)
  bo_p = jnp.zeros((1, 128), f32).at[0, :NC_OUT].set(bo)

  init_call = pl.pallas_call(
      _init_body,
      grid=(NP // BLK,),
      in_specs=[
          _row_spec(BLK), _full((D, D)), _full((1, D)), _full((1, D)),
          _full((1, D)), _full((1, D)), _full((1, 1)),
      ],
      out_specs=[_row_spec(BLK)] * 3,
      out_shape=[jax.ShapeDtypeStruct((NP, D), f32)] * 3,
  )
  h0, f0, u0 = init_call(feats, wt_in, b_in2, gam[0].reshape(1, D),
                         bet[0].reshape(1, D), wg_row, bg2)

  gru_call = pl.pallas_call(
      _gru_body,
      grid=(NP // BLK,),
      in_specs=[
          _row_spec(BLK),
          pl.BlockSpec((NSC, BLK, D), lambda i: (0, i, 0)),
          _row_spec(BLK),
          _full((D, 3 * D)), _full((D, 3 * D)), _full((D, 3 * D)),
          _full((1, 3 * D)), _full((1, 3 * D)),
          _full((1, D)), _full((1, D)), _full((1, D)), _full((1, 1)),
      ],
      out_specs=[_row_spec(BLK)] * 3,
      out_shape=[jax.ShapeDtypeStruct((NP, D), f32)] * 3,
  )

  (aggp0,) = _sc_edges(f0, src1, dst2, zeros128)
  h1, f1, u1 = gru_call(f0, aggp0, h0, wa, wb, wh, bih, bhh,
                        gam[1].reshape(1, D), bet[1].reshape(1, D),
                        wg_row, bg2)
  (aggp1,) = _sc_edges(f1, src1, dst2, zeros128)
  gru2_call = pl.pallas_call(
      _gru2_body,
      grid=(NP // BLK,),
      in_specs=[
          _row_spec(BLK),
          pl.BlockSpec((NSC, BLK, D), lambda i: (0, i, 0)),
          _row_spec(BLK),
          _full((D, 3 * D)), _full((D, 3 * D)), _full((D, 3 * D)),
          _full((1, 3 * D)), _full((1, 3 * D)),
          _full((1, D)), _full((1, 1)),
      ],
      out_specs=[_row_spec(BLK)] * 2,
      out_shape=[jax.ShapeDtypeStruct((NP, D), f32)] * 2,
  )
  h2, u2 = gru2_call(f1, aggp1, h1, wa, wb, wh, bih, bhh, wg_row, bg2)

  sgp, mgp = _sc_readout(u0, u1, u2, h0, h1, h2, idsoff, ids_p, ninf3,
                         zeros128)
  sgp = sgp.reshape(NSC, NL, BP, D)
  mgp = mgp.reshape(NW, NL, BP, D)

  r1_call = pl.pallas_call(
      _r1_body,
      grid=(NL,),
      in_specs=[
          pl.BlockSpec((NSC, 1, BP, D), lambda i: (0, i, 0, 0)),
          pl.BlockSpec((NW, 1, BP, D), lambda i: (0, i, 0, 0)),
      ],
      out_specs=pl.BlockSpec((1, B, 2 * D), lambda i: (i, 0, 0)),
      out_shape=jax.ShapeDtypeStruct((NL, B, 2 * D), f32),
  )
  seq = r1_call(sgp, mgp)

  r2_call = pl.pallas_call(
      _r2_body,
      grid=(1,),
      in_specs=[
          _full((NL, B, 2 * D)),
          _full((2 * D, 4 * D)), _full((1, 4 * D)),
          _full((2 * D, 4 * D)), _full((D, 4 * D)), _full((1, 4 * D)),
          _full((1, 4 * D)), _full((1, 4 * D)),
          _full((2 * D, 128)), _full((1, 128)),
      ],
      out_specs=_full((B, 128)),
      out_shape=jax.ShapeDtypeStruct((B, 128), f32),
  )
  out = r2_call(seq, wif, bif, wib, whb, bib, bhb, bhf, wo_p, bo_p)
  return out[:, :NC_OUT]
